# baseline (speedup 1.0000x reference)
import jax
import jax.numpy as jnp
from jax import lax
from jax.experimental import pallas as pl
from jax.experimental.pallas import tpu as pltpu
from jax.experimental.pallas import tpu_sc as plsc

TOKEN_ID = 31999
R = 16384
H = 2048
NT = 512
NW = 32
ROWS_W = R // NW
CHUNK = 16
NCHUNK = ROWS_W // CHUNK
NBUF = 3
L = 16


def _sc_body(emb_hbm, ids_hbm, t_hbm, w_hbm, b_hbm, out_hbm,
             buf0, buf1, buf2, ids_v, t_v, w_v, b_v,
             t_smem, prefix_smem,
             sem_in0, sem_in1, sem_in2, sem_out0, sem_out1, sem_out2):
    nc = 2
    wid = lax.axis_index("s") * nc + lax.axis_index("c")
    wbase = wid * ROWS_W
    lanes = lax.iota(jnp.int32, L)

    pltpu.sync_copy(ids_hbm, ids_v)
    pltpu.sync_copy(t_hbm, t_v)
    pltpu.sync_copy(w_hbm, w_v)
    pltpu.sync_copy(b_hbm, b_v)

    bufs = (buf0, buf1, buf2)
    sems_in = (sem_in0, sem_in1, sem_in2)
    sems_out = (sem_out0, sem_out1, sem_out2)

    def load(c, bi):
        pltpu.make_async_copy(
            emb_hbm.at[pl.ds(wbase + c * CHUNK, CHUNK)], bufs[bi], sems_in[bi]
        ).start()

    def wait_load(bi):
        pltpu.make_async_copy(
            emb_hbm.at[pl.ds(wbase, CHUNK)], bufs[bi], sems_in[bi]
        ).wait()

    def store(c, bi):
        pltpu.make_async_copy(
            bufs[bi], out_hbm.at[pl.ds(wbase + c * CHUNK, CHUNK)], sems_out[bi]
        ).start()

    def wait_store(bi):
        pltpu.make_async_copy(
            bufs[bi], out_hbm.at[pl.ds(wbase, CHUNK)], sems_out[bi]
        ).wait()

    load(0, 0)
    load(1, 1)

    for v in range(NT // L):
        tv = (t_v[pl.ds(v * L, L)] - 1175.0) * (1.0 / 2350.0)
        for l in range(L):
            t_smem[v * L + l] = jnp.sum(jnp.where(lanes == l, tv, 0.0))

    def pass_a(i, cnt):
        v = ids_v[pl.ds(i * L, L)]
        return cnt + jnp.sum((v == TOKEN_ID).astype(jnp.int32))

    run = lax.fori_loop(0, wid * NCHUNK, pass_a, jnp.int32(0))

    for v in range(NCHUNK):
        prefix_smem[v] = run
        mvec = ids_v[pl.ds(wbase + v * L, L)] == TOKEN_ID
        run = run + jnp.sum(mvec.astype(jnp.int32))

    def patch(c, bi):
        mvec = ids_v[pl.ds(wbase + c * L, L)] == TOKEN_ID
        csum = plsc.cumsum(mvec.astype(jnp.int32))
        gvec = prefix_smem[c] + csum - 1

        def cond(m):
            return jnp.any(m)

        def body(m):
            lane = jnp.max(plsc.all_reduce_ffs(m))
            g = jnp.max(jnp.where(lanes == lane, gvec, -1))
            t_s = t_smem[g]
            row_idx = jnp.full((L,), lane, jnp.int32)
            for h in range(H // L):
                vals = t_s * w_v[pl.ds(h * L, L)] + b_v[pl.ds(h * L, L)]
                plsc.store_scatter(bufs[bi], [row_idx, h * L + lanes], vals)
            return m & (lanes != lane)

        lax.while_loop(cond, body, mvec)

    def ring_step(g, carry):
        for bi in range(NBUF):
            c = g * NBUF + bi
            cc = jnp.int32(c)

            @pl.when(cc < NCHUNK)
            def _step():
                wait_load(bi)
                patch(cc, bi)
                store(cc, bi)
                bj = (bi + 2) % NBUF

                @pl.when(cc + 2 < NCHUNK)
                def _reload():
                    @pl.when(cc >= 1)
                    def _drain():
                        wait_store(bj)

                    load(cc + 2, bj)

        return carry

    lax.fori_loop(0, (NCHUNK + NBUF - 1) // NBUF, ring_step, jnp.int32(0))
    wait_store((NCHUNK - 3) % NBUF)
    wait_store((NCHUNK - 2) % NBUF)
    wait_store((NCHUNK - 1) % NBUF)


@jax.jit
def _sc_call(emb2, ids1, t, w1, b1):
    mesh = plsc.VectorSubcoreMesh(core_axis_name="c", subcore_axis_name="s")
    kfn = pl.kernel(
        _sc_body,
        out_type=jax.ShapeDtypeStruct((R, H), jnp.float32),
        mesh=mesh,
        compiler_params=pltpu.CompilerParams(needs_layout_passes=False),
        scratch_types=[
            pltpu.VMEM((CHUNK, H), jnp.float32),
            pltpu.VMEM((CHUNK, H), jnp.float32),
            pltpu.VMEM((CHUNK, H), jnp.float32),
            pltpu.VMEM((R,), jnp.int32),
            pltpu.VMEM((NT,), jnp.float32),
            pltpu.VMEM((H,), jnp.float32),
            pltpu.VMEM((H,), jnp.float32),
            pltpu.SMEM((NT,), jnp.float32),
            pltpu.SMEM((NCHUNK,), jnp.int32),
            pltpu.SemaphoreType.DMA,
            pltpu.SemaphoreType.DMA,
            pltpu.SemaphoreType.DMA,
            pltpu.SemaphoreType.DMA,
            pltpu.SemaphoreType.DMA,
            pltpu.SemaphoreType.DMA,
        ],
    )
    return kfn(emb2, ids1, t, w1, b1)


def kernel(inputs_embeds, input_ids, t_indices, W, b):
    B, S, Hd = inputs_embeds.shape
    emb2 = inputs_embeds.reshape(B * S, Hd)
    ids1 = input_ids.reshape(B * S)
    out = _sc_call(emb2, ids1, t_indices, W.reshape(Hd), b)
    return out.reshape(B, S, Hd)


# speedup vs baseline: 1.0207x; 1.0207x over previous
import jax
import jax.numpy as jnp
from jax import lax
from jax.experimental import pallas as pl
from jax.experimental.pallas import tpu as pltpu
from jax.experimental.pallas import tpu_sc as plsc

TOKEN_ID = 31999
R = 16384
H = 2048
NT = 512
NW = 32
ROWS_W = R // NW
CHUNK = 16
NCHUNK = ROWS_W // CHUNK
NA = 22
NB = NCHUNK - NA
NBUF = 2
L = 16
NSUB = 16


def _sc_body(emb_hbm, ids_hbm, t_hbm, w_hbm, b_hbm, out_hbm,
             buf0, buf1, ids_v, t_v, w_v, b_v, rowbuf, spmem,
             t_smem, prefix_smem,
             sem_in0, sem_in1, sem_out0, sem_out1,
             bsem_in0, bsem_out0):
    nc = 2
    sid = lax.axis_index("s")
    wid = sid * nc + lax.axis_index("c")
    wbase = wid * ROWS_W
    lanes = lax.iota(jnp.int32, L)

    pltpu.sync_copy(ids_hbm, ids_v)
    pltpu.sync_copy(t_hbm, t_v)
    pltpu.sync_copy(w_hbm, w_v)
    pltpu.sync_copy(b_hbm, b_v)

    bufs = (buf0, buf1)
    sems_in = (sem_in0, sem_in1)
    sems_out = (sem_out0, sem_out1)
    bsems_in = (bsem_in0,)
    bsems_out = (bsem_out0,)

    def a_load(c, bi):
        pltpu.make_async_copy(
            emb_hbm.at[pl.ds(wbase + c * CHUNK, CHUNK)], bufs[bi], sems_in[bi]
        ).start()

    def a_wait_load(bi):
        pltpu.make_async_copy(
            emb_hbm.at[pl.ds(wbase, CHUNK)], bufs[bi], sems_in[bi]
        ).wait()

    def a_store(c, bi):
        pltpu.make_async_copy(
            bufs[bi], out_hbm.at[pl.ds(wbase + c * CHUNK, CHUNK)], sems_out[bi]
        ).start()

    def a_wait_store(bi):
        pltpu.make_async_copy(
            bufs[bi], out_hbm.at[pl.ds(wbase, CHUNK)], sems_out[bi]
        ).wait()

    def b_buf(bi):
        return spmem.at[sid]

    def b_load(c, bi):
        pltpu.make_async_copy(
            emb_hbm.at[pl.ds(wbase + c * CHUNK, CHUNK)], b_buf(bi), bsems_in[bi]
        ).start()

    def b_wait_load(bi):
        pltpu.make_async_copy(
            emb_hbm.at[pl.ds(wbase, CHUNK)], b_buf(bi), bsems_in[bi]
        ).wait()

    def b_store(c, bi):
        pltpu.make_async_copy(
            b_buf(bi), out_hbm.at[pl.ds(wbase + c * CHUNK, CHUNK)], bsems_out[bi]
        ).start()

    def b_wait_store(bi):
        pltpu.make_async_copy(
            b_buf(bi), out_hbm.at[pl.ds(wbase, CHUNK)], bsems_out[bi]
        ).wait()

    a_load(0, 0)
    a_load(1, 1)
    b_load(NA, 0)

    for v in range(NT // L):
        tv = (t_v[pl.ds(v * L, L)] - 1175.0) * (1.0 / 2350.0)
        for l in range(L):
            t_smem[v * L + l] = jnp.sum(jnp.where(lanes == l, tv, 0.0))

    def pass_a(i, cnt):
        v = ids_v[pl.ds(i * L, L)]
        return cnt + jnp.sum((v == TOKEN_ID).astype(jnp.int32))

    run = lax.fori_loop(0, wid * NCHUNK, pass_a, jnp.int32(0))

    for v in range(NCHUNK):
        prefix_smem[v] = run
        mvec = ids_v[pl.ds(wbase + v * L, L)] == TOKEN_ID
        run = run + jnp.sum(mvec.astype(jnp.int32))

    def match_info(c):
        mvec = ids_v[pl.ds(wbase + c * L, L)] == TOKEN_ID
        csum = plsc.cumsum(mvec.astype(jnp.int32))
        gvec = prefix_smem[c] + csum - 1
        return mvec, gvec

    def patch_a(c, bi):
        mvec, gvec = match_info(c)

        def cond(m):
            return jnp.any(m)

        def body(m):
            lane = jnp.max(plsc.all_reduce_ffs(m))
            g = jnp.max(jnp.where(lanes == lane, gvec, -1))
            t_s = t_smem[g]
            row_idx = jnp.full((L,), lane, jnp.int32)
            for h in range(H // L):
                vals = t_s * w_v[pl.ds(h * L, L)] + b_v[pl.ds(h * L, L)]
                plsc.store_scatter(bufs[bi], [row_idx, h * L + lanes], vals)
            return m & (lanes != lane)

        lax.while_loop(cond, body, mvec)

    def patch_b(c, bi):
        mvec, gvec = match_info(c)

        def cond(m):
            return jnp.any(m)

        def body(m):
            lane = jnp.max(plsc.all_reduce_ffs(m))
            g = jnp.max(jnp.where(lanes == lane, gvec, -1))
            t_s = t_smem[g]
            for h in range(H // L):
                rowbuf[0, pl.ds(h * L, L)] = (
                    t_s * w_v[pl.ds(h * L, L)] + b_v[pl.ds(h * L, L)])
            pltpu.sync_copy(rowbuf, b_buf(bi).at[pl.ds(lane, 1)])
            return m & (lanes != lane)

        lax.while_loop(cond, body, mvec)

    def ring_step(g, carry):
        for bi in range(NBUF):
            cca = jnp.int32(g * NBUF + bi)

            @pl.when(cca < NA)
            def _stepa():
                a_wait_load(bi)
                patch_a(cca, bi)
                a_store(cca, bi)

                @pl.when(cca + NBUF < NA)
                def _reload():
                    a_wait_store(bi)
                    a_load(cca + NBUF, bi)

        cb = jnp.int32(NA + g)

        @pl.when(cb < NCHUNK)
        def _stepb():
            b_wait_load(0)
            patch_b(cb, 0)
            b_store(cb, 0)

            @pl.when(cb + 1 < NCHUNK)
            def _breload():
                b_wait_store(0)
                b_load(cb + 1, 0)

        return carry

    nsteps = max((NA + NBUF - 1) // NBUF, NB)
    lax.fori_loop(0, nsteps, ring_step, jnp.int32(0))
    a_wait_store((NA - 2) % NBUF)
    a_wait_store((NA - 1) % NBUF)
    b_wait_store(0)


@jax.jit
def _sc_call(emb2, ids1, t, w1, b1):
    mesh = plsc.VectorSubcoreMesh(core_axis_name="c", subcore_axis_name="s")
    kfn = pl.kernel(
        _sc_body,
        out_type=jax.ShapeDtypeStruct((R, H), jnp.float32),
        mesh=mesh,
        compiler_params=pltpu.CompilerParams(needs_layout_passes=False),
        scratch_types=[
            pltpu.VMEM((CHUNK, H), jnp.float32),
            pltpu.VMEM((CHUNK, H), jnp.float32),
            pltpu.VMEM((R,), jnp.int32),
            pltpu.VMEM((NT,), jnp.float32),
            pltpu.VMEM((H,), jnp.float32),
            pltpu.VMEM((H,), jnp.float32),
            pltpu.VMEM((1, H), jnp.float32),
            pltpu.VMEM_SHARED((NSUB, CHUNK, H), jnp.float32),
            pltpu.SMEM((NT,), jnp.float32),
            pltpu.SMEM((NCHUNK,), jnp.int32),
            pltpu.SemaphoreType.DMA,
            pltpu.SemaphoreType.DMA,
            pltpu.SemaphoreType.DMA,
            pltpu.SemaphoreType.DMA,
            pltpu.SemaphoreType.DMA,
            pltpu.SemaphoreType.DMA,
        ],
    )
    return kfn(emb2, ids1, t, w1, b1)


def kernel(inputs_embeds, input_ids, t_indices, W, b):
    B, S, Hd = inputs_embeds.shape
    emb2 = inputs_embeds.reshape(B * S, Hd)
    ids1 = input_ids.reshape(B * S)
    out = _sc_call(emb2, ids1, t_indices, W.reshape(Hd), b)
    return out.reshape(B, S, Hd)


# speedup vs baseline: 1.0235x; 1.0027x over previous
import jax
import jax.numpy as jnp
from jax import lax
from jax.experimental import pallas as pl
from jax.experimental.pallas import tpu as pltpu
from jax.experimental.pallas import tpu_sc as plsc

TOKEN_ID = 31999
R = 16384
H = 2048
NT = 512
NW = 32
ROWS_W = R // NW
CHUNK = 16
NCHUNK = ROWS_W // CHUNK
NA = 22
NB = NCHUNK - NA
NBUF = 2
L = 16
NSUB = 16


def _sc_body(emb_hbm, ids_hbm, t_hbm, w_hbm, b_hbm, out_hbm,
             buf0, buf1, ids_v, t_v, w_v, b_v, rowbuf, spmem,
             t_smem, prefix_smem,
             sem_in0, sem_in1, sem_out0, sem_out1,
             bsem_in0, bsem_out0):
    nc = 2
    sid = lax.axis_index("s")
    wid = sid * nc + lax.axis_index("c")
    wbase = wid * ROWS_W
    lanes = lax.iota(jnp.int32, L)

    pltpu.sync_copy(ids_hbm, ids_v)
    pltpu.sync_copy(t_hbm, t_v)
    pltpu.sync_copy(w_hbm, w_v)
    pltpu.sync_copy(b_hbm, b_v)

    bufs = (buf0, buf1)
    sems_in = (sem_in0, sem_in1)
    sems_out = (sem_out0, sem_out1)
    bsems_in = (bsem_in0,)
    bsems_out = (bsem_out0,)

    def a_load(c, bi):
        pltpu.make_async_copy(
            emb_hbm.at[pl.ds(wbase + c * CHUNK, CHUNK)], bufs[bi], sems_in[bi]
        ).start()

    def a_wait_load(bi):
        pltpu.make_async_copy(
            emb_hbm.at[pl.ds(wbase, CHUNK)], bufs[bi], sems_in[bi]
        ).wait()

    def a_store(c, bi):
        pltpu.make_async_copy(
            bufs[bi], out_hbm.at[pl.ds(wbase + c * CHUNK, CHUNK)], sems_out[bi]
        ).start()

    def a_wait_store(bi):
        pltpu.make_async_copy(
            bufs[bi], out_hbm.at[pl.ds(wbase, CHUNK)], sems_out[bi]
        ).wait()

    def b_buf(bi):
        return spmem.at[sid]

    def b_load(c, bi):
        pltpu.make_async_copy(
            emb_hbm.at[pl.ds(wbase + c * CHUNK, CHUNK)], b_buf(bi), bsems_in[bi]
        ).start()

    def b_wait_load(bi):
        pltpu.make_async_copy(
            emb_hbm.at[pl.ds(wbase, CHUNK)], b_buf(bi), bsems_in[bi]
        ).wait()

    def b_store(c, bi):
        pltpu.make_async_copy(
            b_buf(bi), out_hbm.at[pl.ds(wbase + c * CHUNK, CHUNK)], bsems_out[bi]
        ).start()

    def b_wait_store(bi):
        pltpu.make_async_copy(
            b_buf(bi), out_hbm.at[pl.ds(wbase, CHUNK)], bsems_out[bi]
        ).wait()

    a_load(0, 0)
    a_load(1, 1)
    b_load(NA, 0)

    for v in range(NT // L):
        tv = (t_v[pl.ds(v * L, L)] - 1175.0) * (1.0 / 2350.0)
        for l in range(L):
            t_smem[v * L + l] = jnp.sum(jnp.where(lanes == l, tv, 0.0))

    def pass_a(i, cnt):
        acc = jnp.zeros((L,), jnp.int32)
        for u in range(4):
            acc = acc + (ids_v[pl.ds(i * 4 * L + u * L, L)]
                         == TOKEN_ID).astype(jnp.int32)
        return cnt + jnp.sum(acc)

    run = lax.fori_loop(0, wid * NCHUNK // 4, pass_a, jnp.int32(0))

    for v in range(NCHUNK):
        prefix_smem[v] = run
        mvec = ids_v[pl.ds(wbase + v * L, L)] == TOKEN_ID
        run = run + jnp.sum(mvec.astype(jnp.int32))

    def match_info(c):
        mvec = ids_v[pl.ds(wbase + c * L, L)] == TOKEN_ID
        csum = plsc.cumsum(mvec.astype(jnp.int32))
        gvec = prefix_smem[c] + csum - 1
        return mvec, gvec

    def patch_a(c, bi):
        mvec, gvec = match_info(c)

        def cond(m):
            return jnp.any(m)

        def body(m):
            lane = jnp.max(plsc.all_reduce_ffs(m))
            g = jnp.max(jnp.where(lanes == lane, gvec, -1))
            t_s = t_smem[g]
            row_idx = jnp.full((L,), lane, jnp.int32)
            for h in range(H // L):
                vals = t_s * w_v[pl.ds(h * L, L)] + b_v[pl.ds(h * L, L)]
                plsc.store_scatter(bufs[bi], [row_idx, h * L + lanes], vals)
            return m & (lanes != lane)

        lax.while_loop(cond, body, mvec)

    def patch_b(c, bi):
        mvec, gvec = match_info(c)

        def cond(m):
            return jnp.any(m)

        def body(m):
            lane = jnp.max(plsc.all_reduce_ffs(m))
            g = jnp.max(jnp.where(lanes == lane, gvec, -1))
            t_s = t_smem[g]
            for h in range(H // L):
                rowbuf[0, pl.ds(h * L, L)] = (
                    t_s * w_v[pl.ds(h * L, L)] + b_v[pl.ds(h * L, L)])
            pltpu.sync_copy(rowbuf, b_buf(bi).at[pl.ds(lane, 1)])
            return m & (lanes != lane)

        lax.while_loop(cond, body, mvec)

    def ring_step(g, carry):
        for bi in range(NBUF):
            cca = jnp.int32(g * NBUF + bi)

            @pl.when(cca < NA)
            def _stepa():
                a_wait_load(bi)
                patch_a(cca, bi)
                a_store(cca, bi)

                @pl.when(cca + NBUF < NA)
                def _reload():
                    a_wait_store(bi)
                    a_load(cca + NBUF, bi)

        cb = jnp.int32(NA + g)

        @pl.when(cb < NCHUNK)
        def _stepb():
            b_wait_load(0)
            patch_b(cb, 0)
            b_store(cb, 0)

            @pl.when(cb + 1 < NCHUNK)
            def _breload():
                b_wait_store(0)
                b_load(cb + 1, 0)

        return carry

    nsteps = max((NA + NBUF - 1) // NBUF, NB)
    lax.fori_loop(0, nsteps, ring_step, jnp.int32(0))
    a_wait_store((NA - 2) % NBUF)
    a_wait_store((NA - 1) % NBUF)
    b_wait_store(0)


@jax.jit
def _sc_call(emb2, ids1, t, w1, b1):
    mesh = plsc.VectorSubcoreMesh(core_axis_name="c", subcore_axis_name="s")
    kfn = pl.kernel(
        _sc_body,
        out_type=jax.ShapeDtypeStruct((R, H), jnp.float32),
        mesh=mesh,
        compiler_params=pltpu.CompilerParams(needs_layout_passes=False),
        scratch_types=[
            pltpu.VMEM((CHUNK, H), jnp.float32),
            pltpu.VMEM((CHUNK, H), jnp.float32),
            pltpu.VMEM((R,), jnp.int32),
            pltpu.VMEM((NT,), jnp.float32),
            pltpu.VMEM((H,), jnp.float32),
            pltpu.VMEM((H,), jnp.float32),
            pltpu.VMEM((1, H), jnp.float32),
            pltpu.VMEM_SHARED((NSUB, CHUNK, H), jnp.float32),
            pltpu.SMEM((NT,), jnp.float32),
            pltpu.SMEM((NCHUNK,), jnp.int32),
            pltpu.SemaphoreType.DMA,
            pltpu.SemaphoreType.DMA,
            pltpu.SemaphoreType.DMA,
            pltpu.SemaphoreType.DMA,
            pltpu.SemaphoreType.DMA,
            pltpu.SemaphoreType.DMA,
        ],
    )
    return kfn(emb2, ids1, t, w1, b1)


def kernel(inputs_embeds, input_ids, t_indices, W, b):
    B, S, Hd = inputs_embeds.shape
    emb2 = inputs_embeds.reshape(B * S, Hd)
    ids1 = input_ids.reshape(B * S)
    out = _sc_call(emb2, ids1, t_indices, W.reshape(Hd), b)
    return out.reshape(B, S, Hd)
